# Initial kernel scaffold; baseline (speedup 1.0000x reference)
import jax
import jax.numpy as jnp
from jax.experimental import pallas as pl


def kernel(x, router_w, router_b, gate_w, up_w, down_w):
    raise NotImplementedError("write your pallas kernel here")


# speedup vs baseline: 1.3318x; 1.3318x over previous
import functools

import jax
import jax.numpy as jnp
from jax.experimental import pallas as pl


def _ffn_body(x_ref, comb_ref, gw_ref, uw_ref, dw_ref, out_ref, *, n_experts):
    e = pl.program_id(0)
    i = pl.program_id(1)

    @pl.when((e == 0) & (i == 0))
    def _init():
        out_ref[...] = jnp.zeros_like(out_ref)

    xb = x_ref[...]
    gw = gw_ref[0]
    uw = uw_ref[0]
    dw = dw_ref[0]

    dn = (((1,), (1,)), ((), ()))
    g = jax.lax.dot_general(xb, gw, dn, preferred_element_type=jnp.float32)
    u = jax.lax.dot_general(xb, uw, dn, preferred_element_type=jnp.float32)
    h = jax.nn.silu(g) * u

    col = jax.lax.broadcasted_iota(jnp.int32, comb_ref.shape, 1)
    c = jnp.sum(jnp.where(col == e, comb_ref[...], 0.0), axis=1, keepdims=True)
    h = h * c

    contrib = jax.lax.dot_general(h, dw, dn, preferred_element_type=jnp.float32)
    out_ref[...] += contrib


def kernel(x, router_w, router_b, gate_w, up_w, down_w):
    B, S, D = x.shape
    E, DI, _ = gate_w.shape
    N = B * S
    xf = x.reshape(N, D)

    logits = xf @ router_w.T + router_b
    probs = jax.nn.softmax(logits, axis=-1)
    topk_p, topk_i = jax.lax.top_k(probs, 2)
    topk_p = topk_p / jnp.sum(topk_p, axis=-1, keepdims=True)
    combine = jnp.sum(
        topk_p[..., None] * jax.nn.one_hot(topk_i, E, dtype=x.dtype), axis=-2
    )

    Ki = min(256, DI)
    NI = DI // Ki

    out = pl.pallas_call(
        functools.partial(_ffn_body, n_experts=E),
        grid=(E, NI),
        in_specs=[
            pl.BlockSpec((N, D), lambda e, i: (0, 0)),
            pl.BlockSpec((N, E), lambda e, i: (0, 0)),
            pl.BlockSpec((1, Ki, D), lambda e, i: (e, i, 0)),
            pl.BlockSpec((1, Ki, D), lambda e, i: (e, i, 0)),
            pl.BlockSpec((1, D, Ki), lambda e, i: (e, 0, i)),
        ],
        out_specs=pl.BlockSpec((N, D), lambda e, i: (0, 0)),
        out_shape=jax.ShapeDtypeStruct((N, D), jnp.float32),
    )(xf, combine, gate_w, up_w, down_w)

    return out.reshape(B, S, D)
